# Initial kernel scaffold; baseline (speedup 1.0000x reference)
import functools

import jax
import jax.numpy as jnp
from jax import lax
from jax.experimental import pallas as pl
from jax.experimental.pallas import tpu as pltpu
from jax.experimental.pallas import tpu_sc as plsc

N = 10000
E = 320000
D = 128
ALPHA = 0.05
K_PROP = 10

NC = 2
NS = 16
NW = NC * NS

N_PAD = 10240
E_PAD = 327680
CH = 128
N_CHUNKS = E_PAD // CH
CPS = N_CHUNKS // NC
CPT = CPS // NS
ROWS_T = N_PAD // NS
ZR = 64
EMB_CH = 64
EMB_PT = N_PAD // NW

f32 = jnp.float32
i32 = jnp.int32

_mesh = plsc.VectorSubcoreMesh(core_axis_name="c", subcore_axis_name="s")


@functools.partial(
    pl.kernel,
    mesh=_mesh,
    out_type=(
        jax.ShapeDtypeStruct((NC, N_PAD, 16), f32),
        jax.ShapeDtypeStruct((NC, N_PAD, 16), f32),
        jax.ShapeDtypeStruct((N_PAD, D), f32),
    ),
    scratch_types=[
        pltpu.VMEM((CH,), i32),
        pltpu.VMEM((CH, 16), f32),
        pltpu.VMEM((ZR, 16), f32),
        pltpu.VMEM((EMB_CH,), i32),
        pltpu.VMEM((EMB_CH, D), f32),
        pltpu.VMEM_SHARED((N_PAD, 16), f32),
        pltpu.VMEM_SHARED((N_PAD, 16), f32),
        pltpu.SemaphoreType.DMA,
    ],
)
def _sc_setup(h_hbm, src_hbm, dst_hbm, table_hbm,
              dego_hbm, degi_hbm, emb_hbm,
              eidx, ones_v, z16, hidx, rows_v, acc_o, acc_i, sem):
    cidx = lax.axis_index("c")
    sid = lax.axis_index("s")
    wid = cidx * NS + sid

    zeros16 = jnp.zeros((16,), f32)
    ones16 = jnp.ones((16,), f32)

    @pl.loop(0, CH)
    def _(r):
        ones_v[r, :] = ones16

    @pl.loop(0, ZR)
    def _(r):
        z16[r, :] = zeros16

    @pl.loop(0, ROWS_T // ZR)
    def _(t):
        base = sid * ROWS_T + t * ZR
        pltpu.sync_copy(z16, acc_o.at[pl.ds(base, ZR)])
        pltpu.sync_copy(z16, acc_i.at[pl.ds(base, ZR)])

    plsc.subcore_barrier()

    @pl.loop(0, CPT)
    def _(t):
        base = (cidx * CPS + sid + t * NS) * CH
        pltpu.sync_copy(src_hbm.at[pl.ds(base, CH)], eidx)
        pltpu.sync_copy(ones_v, acc_o.at[eidx], add=True)
        pltpu.sync_copy(dst_hbm.at[pl.ds(base, CH)], eidx)
        pltpu.sync_copy(ones_v, acc_i.at[eidx], add=True)

    @pl.loop(0, EMB_PT // EMB_CH)
    def _(t):
        base = wid * EMB_PT + t * EMB_CH
        pltpu.sync_copy(h_hbm.at[pl.ds(base, EMB_CH)], hidx)
        pltpu.async_copy(table_hbm.at[hidx], rows_v, sem).wait()
        pltpu.sync_copy(rows_v, emb_hbm.at[pl.ds(base, EMB_CH)])

    plsc.subcore_barrier()

    @pl.loop(0, ROWS_T // ZR)
    def _(t):
        base = sid * ROWS_T + t * ZR
        pltpu.sync_copy(acc_o.at[pl.ds(base, ZR)], z16)
        pltpu.sync_copy(z16, dego_hbm.at[cidx, pl.ds(base, ZR)])
        pltpu.sync_copy(acc_i.at[pl.ds(base, ZR)], z16)
        pltpu.sync_copy(z16, degi_hbm.at[cidx, pl.ds(base, ZR)])


@functools.partial(
    pl.kernel,
    mesh=_mesh,
    out_type=jax.ShapeDtypeStruct((NC, N_PAD, D), f32),
    scratch_types=[
        pltpu.VMEM((CH,), i32),
        pltpu.VMEM((CH,), i32),
        pltpu.VMEM((CH, D), f32),
        pltpu.VMEM((ZR, D), f32),
        pltpu.VMEM_SHARED((N_PAD, D), f32),
        pltpu.SemaphoreType.DMA,
    ],
)
def _sc_prop(y_hbm, src_hbm, dst_hbm, sp_hbm,
             sidx, didx, rows, zb, acc, sem):
    cidx = lax.axis_index("c")
    sid = lax.axis_index("s")

    zeros16 = jnp.zeros((16,), f32)

    @pl.loop(0, ZR)
    def _(r):
        @pl.loop(0, D, step=16)
        def _(c):
            zb[r, pl.ds(c, 16)] = zeros16

    @pl.loop(0, ROWS_T // ZR)
    def _(t):
        pltpu.sync_copy(zb, acc.at[pl.ds(sid * ROWS_T + t * ZR, ZR)])

    plsc.subcore_barrier()

    @pl.loop(0, CPT)
    def _(t):
        base = (cidx * CPS + sid + t * NS) * CH
        pltpu.sync_copy(src_hbm.at[pl.ds(base, CH)], sidx)
        pltpu.sync_copy(dst_hbm.at[pl.ds(base, CH)], didx)
        pltpu.async_copy(y_hbm.at[sidx], rows, sem).wait()
        pltpu.sync_copy(rows, acc.at[didx], add=True)

    plsc.subcore_barrier()

    @pl.loop(0, ROWS_T // ZR)
    def _(t):
        base = sid * ROWS_T + t * ZR
        pltpu.sync_copy(acc.at[pl.ds(base, ZR)], zb)
        pltpu.sync_copy(zb, sp_hbm.at[cidx, pl.ds(base, ZR)])


def _prep_body(dop, dip, emb, g0, cm, inv):
    do_p = dop[...]
    di_p = dip[...]
    deg_o = (do_p[0] + do_p[1])[:, 0:1]
    deg_i = (di_p[0] + di_p[1])[:, 0:1]
    no = lax.rsqrt(jnp.where(deg_o > 0, deg_o, 1.0))
    ni = lax.rsqrt(jnp.where(deg_i > 0, deg_i, 1.0))
    mask = (lax.broadcasted_iota(i32, (N_PAD, 1), 0) < N).astype(f32)
    g0[...] = emb[...] * (no * mask)
    cm[...] = (1.0 - ALPHA) * no * ni * mask
    inv[...] = mask / no


_tc_prep = pl.pallas_call(
    _prep_body,
    out_shape=(
        jax.ShapeDtypeStruct((N_PAD, D), f32),
        jax.ShapeDtypeStruct((N_PAD, 1), f32),
        jax.ShapeDtypeStruct((N_PAD, 1), f32),
    ),
)


def _update_body(sp, cm, g0, y):
    s = sp[0] + sp[1]
    y[...] = cm[...] * s + ALPHA * g0[...]


_tc_update = pl.pallas_call(
    _update_body,
    out_shape=jax.ShapeDtypeStruct((N_PAD, D), f32),
)


def _mlp_body(y, inv, w1, b1, w2, b2, w3, b3, out):
    x = y[...] * inv[...]
    h1 = jnp.maximum(jnp.dot(x, w1[...], preferred_element_type=f32) + b1[...], 0.0)
    h2 = jnp.maximum(jnp.dot(h1, w2[...], preferred_element_type=f32) + b2[...], 0.0)
    out[...] = jnp.dot(h2, w3[...], preferred_element_type=f32) + b3[...]


_tc_mlp = pl.pallas_call(
    _mlp_body,
    out_shape=jax.ShapeDtypeStruct((N_PAD, D), f32),
)


def kernel(h, edge_index, table, W1, b1, W2, b2, W3, b3):
    h_pad = jnp.zeros((N_PAD,), i32).at[:N].set(h.astype(i32))
    src_pad = jnp.full((E_PAD,), N, i32).at[:E].set(edge_index[0].astype(i32))
    dst_pad = jnp.full((E_PAD,), N, i32).at[:E].set(edge_index[1].astype(i32))

    dego, degi, emb = _sc_setup(h_pad, src_pad, dst_pad, table)
    g0, cm, inv = _tc_prep(dego, degi, emb)
    y = g0
    for _ in range(K_PROP):
        sp = _sc_prop(y, src_pad, dst_pad)
        y = _tc_update(sp, cm, g0)
    out = _tc_mlp(y, inv, W1, b1, W2, b2, W3, b3)
    return out[:N]


# speedup vs baseline: 2.9275x; 2.9275x over previous
import functools

import jax
import jax.numpy as jnp
from jax import lax
from jax.experimental import pallas as pl
from jax.experimental.pallas import tpu as pltpu
from jax.experimental.pallas import tpu_sc as plsc

N = 10000
E = 320000
D = 128
ALPHA = 0.05
K_PROP = 10

NC = 2
NS = 16
NW = NC * NS

N_PAD = 10240
E_PAD = 327680
CH = 128
N_CHUNKS = E_PAD // CH
CPS = N_CHUNKS // NC
CPT = CPS // NS
ROWS_T = N_PAD // NS
ZR = 64
EMB_CH = 64
EMB_PT = N_PAD // NW

f32 = jnp.float32
i32 = jnp.int32

_mesh = plsc.VectorSubcoreMesh(core_axis_name="c", subcore_axis_name="s")


@functools.partial(
    pl.kernel,
    mesh=_mesh,
    out_type=(
        jax.ShapeDtypeStruct((NC, N_PAD, D), f32),
        jax.ShapeDtypeStruct((NC, N_PAD, D), f32),
        jax.ShapeDtypeStruct((N_PAD, D), f32),
    ),
    scratch_types=[
        pltpu.VMEM((CH,), i32),
        pltpu.VMEM((CH, D), f32),
        pltpu.VMEM((ZR, D), f32),
        pltpu.VMEM((EMB_CH,), i32),
        pltpu.VMEM((EMB_CH, D), f32),
        pltpu.VMEM_SHARED((N_PAD, D), f32),
        pltpu.SemaphoreType.DMA,
    ],
)
def _sc_setup(h_hbm, src_hbm, dst_hbm, table_hbm,
              dego_hbm, degi_hbm, emb_hbm,
              eidx, ones_v, zb, hidx, rows_v, acc, sem):
    cidx = lax.axis_index("c")
    sid = lax.axis_index("s")
    wid = cidx * NS + sid

    zeros16 = jnp.zeros((16,), f32)
    ones16 = jnp.ones((16,), f32)

    @pl.loop(0, CH)
    def _(r):
        @pl.loop(0, D, step=16)
        def _(c):
            ones_v[r, pl.ds(c, 16)] = ones16

    @pl.loop(0, ZR)
    def _(r):
        @pl.loop(0, D, step=16)
        def _(c):
            zb[r, pl.ds(c, 16)] = zeros16

    @pl.loop(0, ROWS_T // ZR)
    def _(t):
        pltpu.sync_copy(zb, acc.at[pl.ds(sid * ROWS_T + t * ZR, ZR)])

    plsc.subcore_barrier()

    @pl.loop(0, CPT)
    def _(t):
        base = (cidx * CPS + sid + t * NS) * CH
        pltpu.sync_copy(src_hbm.at[pl.ds(base, CH)], eidx)
        pltpu.sync_copy(ones_v, acc.at[eidx], add=True)

    @pl.loop(0, EMB_PT // EMB_CH)
    def _(t):
        base = wid * EMB_PT + t * EMB_CH
        pltpu.sync_copy(h_hbm.at[pl.ds(base, EMB_CH)], hidx)
        pltpu.async_copy(table_hbm.at[hidx], rows_v, sem).wait()
        pltpu.sync_copy(rows_v, emb_hbm.at[pl.ds(base, EMB_CH)])

    plsc.subcore_barrier()

    @pl.loop(0, ROWS_T // EMB_CH)
    def _(t):
        base = sid * ROWS_T + t * EMB_CH
        pltpu.sync_copy(acc.at[pl.ds(base, EMB_CH)], rows_v)
        pltpu.sync_copy(rows_v, dego_hbm.at[cidx, pl.ds(base, EMB_CH)])

    @pl.loop(0, ROWS_T // ZR)
    def _(t):
        pltpu.sync_copy(zb, acc.at[pl.ds(sid * ROWS_T + t * ZR, ZR)])

    plsc.subcore_barrier()

    @pl.loop(0, CPT)
    def _(t):
        base = (cidx * CPS + sid + t * NS) * CH
        pltpu.sync_copy(dst_hbm.at[pl.ds(base, CH)], eidx)
        pltpu.sync_copy(ones_v, acc.at[eidx], add=True)

    plsc.subcore_barrier()

    @pl.loop(0, ROWS_T // EMB_CH)
    def _(t):
        base = sid * ROWS_T + t * EMB_CH
        pltpu.sync_copy(acc.at[pl.ds(base, EMB_CH)], rows_v)
        pltpu.sync_copy(rows_v, degi_hbm.at[cidx, pl.ds(base, EMB_CH)])


@functools.partial(
    pl.kernel,
    mesh=_mesh,
    out_type=jax.ShapeDtypeStruct((NC, N_PAD, D), f32),
    scratch_types=[
        pltpu.VMEM((CH,), i32),
        pltpu.VMEM((CH,), i32),
        pltpu.VMEM((CH, D), f32),
        pltpu.VMEM((ZR, D), f32),
        pltpu.VMEM_SHARED((N_PAD, D), f32),
        pltpu.SemaphoreType.DMA,
    ],
)
def _sc_prop(y_hbm, src_hbm, dst_hbm, sp_hbm,
             sidx, didx, rows, zb, acc, sem):
    cidx = lax.axis_index("c")
    sid = lax.axis_index("s")

    zeros16 = jnp.zeros((16,), f32)

    @pl.loop(0, ZR)
    def _(r):
        @pl.loop(0, D, step=16)
        def _(c):
            zb[r, pl.ds(c, 16)] = zeros16

    @pl.loop(0, ROWS_T // ZR)
    def _(t):
        pltpu.sync_copy(zb, acc.at[pl.ds(sid * ROWS_T + t * ZR, ZR)])

    plsc.subcore_barrier()

    @pl.loop(0, CPT)
    def _(t):
        base = (cidx * CPS + sid + t * NS) * CH
        pltpu.sync_copy(src_hbm.at[pl.ds(base, CH)], sidx)
        pltpu.sync_copy(dst_hbm.at[pl.ds(base, CH)], didx)
        pltpu.async_copy(y_hbm.at[sidx], rows, sem).wait()
        pltpu.sync_copy(rows, acc.at[didx], add=True)

    plsc.subcore_barrier()

    @pl.loop(0, ROWS_T // ZR)
    def _(t):
        base = sid * ROWS_T + t * ZR
        pltpu.sync_copy(acc.at[pl.ds(base, ZR)], zb)
        pltpu.sync_copy(zb, sp_hbm.at[cidx, pl.ds(base, ZR)])


def _prep_body(dop, dip, emb, g0, cm, inv):
    do_p = dop[...]
    di_p = dip[...]
    deg_o = (do_p[0] + do_p[1])[:, 0:1]
    deg_i = (di_p[0] + di_p[1])[:, 0:1]
    no = lax.rsqrt(jnp.where(deg_o > 0, deg_o, 1.0))
    ni = lax.rsqrt(jnp.where(deg_i > 0, deg_i, 1.0))
    mask = (lax.broadcasted_iota(i32, (N_PAD, 1), 0) < N).astype(f32)
    g0[...] = emb[...] * (no * mask)
    cm[...] = (1.0 - ALPHA) * no * ni * mask
    inv[...] = mask / no


_tc_prep = pl.pallas_call(
    _prep_body,
    out_shape=(
        jax.ShapeDtypeStruct((N_PAD, D), f32),
        jax.ShapeDtypeStruct((N_PAD, 1), f32),
        jax.ShapeDtypeStruct((N_PAD, 1), f32),
    ),
)


def _update_body(sp, cm, g0, y):
    s = sp[0] + sp[1]
    y[...] = cm[...] * s + ALPHA * g0[...]


_tc_update = pl.pallas_call(
    _update_body,
    out_shape=jax.ShapeDtypeStruct((N_PAD, D), f32),
)


def _mlp_body(y, inv, w1, b1, w2, b2, w3, b3, out):
    x = y[...] * inv[...]
    h1 = jnp.maximum(jnp.dot(x, w1[...], preferred_element_type=f32) + b1[...], 0.0)
    h2 = jnp.maximum(jnp.dot(h1, w2[...], preferred_element_type=f32) + b2[...], 0.0)
    out[...] = jnp.dot(h2, w3[...], preferred_element_type=f32) + b3[...]


_tc_mlp = pl.pallas_call(
    _mlp_body,
    out_shape=jax.ShapeDtypeStruct((N_PAD, D), f32),
)


def kernel(h, edge_index, table, W1, b1, W2, b2, W3, b3):
    h_pad = jnp.zeros((N_PAD,), i32).at[:N].set(h.astype(i32))
    src_pad = jnp.full((E_PAD,), N, i32).at[:E].set(edge_index[0].astype(i32))
    dst_pad = jnp.full((E_PAD,), N, i32).at[:E].set(edge_index[1].astype(i32))

    dego, degi, emb = _sc_setup(h_pad, src_pad, dst_pad, table)
    g0, cm, inv = _tc_prep(dego, degi, emb)
    y = g0
    for _ in range(K_PROP):
        sp = _sc_prop(y, src_pad, dst_pad)
        y = _tc_update(sp, cm, g0)
    out = _tc_mlp(y, inv, W1, b1, W2, b2, W3, b3)
    return out[:N]


# speedup vs baseline: 3.3080x; 1.1300x over previous
import functools

import jax
import jax.numpy as jnp
from jax import lax
from jax.experimental import pallas as pl
from jax.experimental.pallas import tpu as pltpu
from jax.experimental.pallas import tpu_sc as plsc

N = 10000
E = 320000
D = 128
ALPHA = 0.05
K_PROP = 10

NC = 2
NS = 16
NW = NC * NS

N_PAD = 10240
E_PAD = 327680
CH = 128
N_CHUNKS = E_PAD // CH
CPS = N_CHUNKS // NC
CPT = CPS // NS
ROWS_T = N_PAD // NS
ZR = 64
EMB_CH = 64
EMB_PT = N_PAD // NW

f32 = jnp.float32
i32 = jnp.int32

_mesh = plsc.VectorSubcoreMesh(core_axis_name="c", subcore_axis_name="s")


@functools.partial(
    pl.kernel,
    mesh=_mesh,
    out_type=(
        jax.ShapeDtypeStruct((NC, N_PAD, D), f32),
        jax.ShapeDtypeStruct((NC, N_PAD, D), f32),
        jax.ShapeDtypeStruct((N_PAD, D), f32),
    ),
    scratch_types=[
        pltpu.VMEM((CH,), i32),
        pltpu.VMEM((CH, D), f32),
        pltpu.VMEM((ZR, D), f32),
        pltpu.VMEM((EMB_CH,), i32),
        pltpu.VMEM((EMB_CH, D), f32),
        pltpu.VMEM_SHARED((N_PAD, D), f32),
        pltpu.SemaphoreType.DMA,
    ],
)
def _sc_setup(h_hbm, src_hbm, dst_hbm, table_hbm,
              dego_hbm, degi_hbm, emb_hbm,
              eidx, ones_v, zb, hidx, rows_v, acc, sem):
    cidx = lax.axis_index("c")
    sid = lax.axis_index("s")
    wid = cidx * NS + sid

    zeros16 = jnp.zeros((16,), f32)
    ones16 = jnp.ones((16,), f32)

    @pl.loop(0, CH)
    def _(r):
        @pl.loop(0, D, step=16)
        def _(c):
            ones_v[r, pl.ds(c, 16)] = ones16

    @pl.loop(0, ZR)
    def _(r):
        @pl.loop(0, D, step=16)
        def _(c):
            zb[r, pl.ds(c, 16)] = zeros16

    @pl.loop(0, ROWS_T // ZR)
    def _(t):
        pltpu.sync_copy(zb, acc.at[pl.ds(sid * ROWS_T + t * ZR, ZR)])

    plsc.subcore_barrier()

    @pl.loop(0, CPT)
    def _(t):
        k = cidx * CPS + sid + t * NS
        pltpu.sync_copy(src_hbm.at[k], eidx)
        pltpu.sync_copy(ones_v, acc.at[eidx], add=True)

    @pl.loop(0, EMB_PT // EMB_CH)
    def _(t):
        base = wid * EMB_PT + t * EMB_CH
        pltpu.sync_copy(h_hbm.at[pl.ds(base, EMB_CH)], hidx)
        pltpu.async_copy(table_hbm.at[hidx], rows_v, sem).wait()
        pltpu.sync_copy(rows_v, emb_hbm.at[pl.ds(base, EMB_CH)])

    plsc.subcore_barrier()

    @pl.loop(0, ROWS_T // EMB_CH)
    def _(t):
        base = sid * ROWS_T + t * EMB_CH
        pltpu.sync_copy(acc.at[pl.ds(base, EMB_CH)], rows_v)
        pltpu.sync_copy(rows_v, dego_hbm.at[cidx, pl.ds(base, EMB_CH)])

    @pl.loop(0, ROWS_T // ZR)
    def _(t):
        pltpu.sync_copy(zb, acc.at[pl.ds(sid * ROWS_T + t * ZR, ZR)])

    plsc.subcore_barrier()

    @pl.loop(0, CPT)
    def _(t):
        k = cidx * CPS + sid + t * NS
        pltpu.sync_copy(dst_hbm.at[k], eidx)
        pltpu.sync_copy(ones_v, acc.at[eidx], add=True)

    plsc.subcore_barrier()

    @pl.loop(0, ROWS_T // EMB_CH)
    def _(t):
        base = sid * ROWS_T + t * EMB_CH
        pltpu.sync_copy(acc.at[pl.ds(base, EMB_CH)], rows_v)
        pltpu.sync_copy(rows_v, degi_hbm.at[cidx, pl.ds(base, EMB_CH)])


CT = N_CHUNKS // NW


def _drain(hbm_ref, dst_ref, sem):
    pltpu.make_async_copy(hbm_ref, dst_ref, sem).wait()


@functools.partial(
    pl.kernel,
    mesh=_mesh,
    out_type=jax.ShapeDtypeStruct((NC, N_PAD, D), f32),
    scratch_types=[
        pltpu.VMEM((2, CH), i32),
        pltpu.VMEM((2, CH), i32),
        pltpu.VMEM((2, CH, D), f32),
        pltpu.VMEM_SHARED((N_PAD, D), f32),
        pltpu.SemaphoreType.DMA((2,)),
        pltpu.SemaphoreType.DMA((2,)),
    ],
)
def _sc_prop(y_hbm, src2_hbm, dst2_hbm, sp_hbm,
             sidx, didx, rows, acc, gsem, isem):
    cidx = lax.axis_index("c")
    sid = lax.axis_index("s")
    wid = cidx * NS + sid
    c0 = wid * CT

    zeros16 = jnp.zeros((16,), f32)

    @pl.loop(0, CH)
    def _(r):
        @pl.loop(0, D, step=16)
        def _(c):
            rows[0, r, pl.ds(c, 16)] = zeros16

    @pl.loop(0, ROWS_T // CH)
    def _(t):
        pltpu.sync_copy(rows.at[0], acc.at[pl.ds(sid * ROWS_T + t * CH, CH)])

    plsc.subcore_barrier()

    def idx_load(chunk, b):
        pltpu.async_copy(src2_hbm.at[c0 + chunk], sidx.at[b], isem.at[b])
        pltpu.async_copy(dst2_hbm.at[c0 + chunk], didx.at[b], isem.at[b])

    def idx_wait(b):
        _drain(src2_hbm.at[0], sidx.at[b], isem.at[b])
        _drain(dst2_hbm.at[0], didx.at[b], isem.at[b])

    def gather(b):
        pltpu.async_copy(y_hbm.at[sidx.at[b]], rows.at[b], gsem.at[b])

    def gather_wait(b):
        _drain(y_hbm.at[pl.ds(0, CH)], rows.at[b], gsem.at[b])

    def scatter(b):
        pltpu.sync_copy(rows.at[b], acc.at[didx.at[b]], add=True)

    idx_load(0, 0)
    idx_load(1, 1)
    idx_wait(0)
    gather(0)

    @pl.loop(0, CT - 2, step=2)
    def _(t):
        for b in (0, 1):
            o = b ^ 1
            idx_wait(o)
            gather(o)
            gather_wait(b)
            scatter(b)
            idx_load(t + b + 2, b)

    idx_wait(1)
    gather(1)
    gather_wait(0)
    scatter(0)
    gather_wait(1)
    scatter(1)

    plsc.subcore_barrier()

    @pl.loop(0, ROWS_T // CH)
    def _(t):
        base = sid * ROWS_T + t * CH
        pltpu.sync_copy(acc.at[pl.ds(base, CH)], rows.at[0])
        pltpu.sync_copy(rows.at[0], sp_hbm.at[cidx, pl.ds(base, CH)])


def _prep_body(dop, dip, emb, g0, cm, inv):
    do_p = dop[...]
    di_p = dip[...]
    deg_o = (do_p[0] + do_p[1])[:, 0:1]
    deg_i = (di_p[0] + di_p[1])[:, 0:1]
    no = lax.rsqrt(jnp.where(deg_o > 0, deg_o, 1.0))
    ni = lax.rsqrt(jnp.where(deg_i > 0, deg_i, 1.0))
    mask = (lax.broadcasted_iota(i32, (N_PAD, 1), 0) < N).astype(f32)
    g0[...] = emb[...] * (no * mask)
    cm[...] = (1.0 - ALPHA) * no * ni * mask
    inv[...] = mask / no


_tc_prep = pl.pallas_call(
    _prep_body,
    out_shape=(
        jax.ShapeDtypeStruct((N_PAD, D), f32),
        jax.ShapeDtypeStruct((N_PAD, 1), f32),
        jax.ShapeDtypeStruct((N_PAD, 1), f32),
    ),
)


def _update_body(sp, cm, g0, y):
    s = sp[0] + sp[1]
    y[...] = cm[...] * s + ALPHA * g0[...]


_tc_update = pl.pallas_call(
    _update_body,
    out_shape=jax.ShapeDtypeStruct((N_PAD, D), f32),
)


def _mlp_body(y, inv, w1, b1, w2, b2, w3, b3, out):
    x = y[...] * inv[...]
    h1 = jnp.maximum(jnp.dot(x, w1[...], preferred_element_type=f32) + b1[...], 0.0)
    h2 = jnp.maximum(jnp.dot(h1, w2[...], preferred_element_type=f32) + b2[...], 0.0)
    out[...] = jnp.dot(h2, w3[...], preferred_element_type=f32) + b3[...]


_tc_mlp = pl.pallas_call(
    _mlp_body,
    out_shape=jax.ShapeDtypeStruct((N_PAD, D), f32),
)


def kernel(h, edge_index, table, W1, b1, W2, b2, W3, b3):
    h_pad = jnp.zeros((N_PAD,), i32).at[:N].set(h.astype(i32))
    src_flat = jnp.full((E_PAD,), N, i32).at[:E].set(edge_index[0].astype(i32))
    dst_flat = jnp.full((E_PAD,), N, i32).at[:E].set(edge_index[1].astype(i32))
    src128 = src_flat.reshape(N_CHUNKS, CH)
    dst128 = dst_flat.reshape(N_CHUNKS, CH)

    dego, degi, emb = _sc_setup(h_pad, src128, dst128, table)
    g0, cm, inv = _tc_prep(dego, degi, emb)
    y = g0
    for _ in range(K_PROP):
        sp = _sc_prop(y, src128, dst128)
        y = _tc_update(sp, cm, g0)
    out = _tc_mlp(y, inv, W1, b1, W2, b2, W3, b3)
    return out[:N]
